# Initial kernel scaffold; baseline (speedup 1.0000x reference)
import functools

import jax
import jax.numpy as jnp
from jax import lax
from jax.experimental import pallas as pl
from jax.experimental.pallas import tpu as pltpu
from jax.experimental.pallas import tpu_sc as plsc

_NC = 2
_NS = 16
_NW = _NC * _NS
_CHUNK = 128


def _merge_body(main_ref, a_ref, bt_ref, out_ref):
    out_ref[...] = main_ref[...] + jnp.dot(
        a_ref[...], bt_ref[...], preferred_element_type=jnp.float32
    )


def _merged_table(main_weight, lora_a_scaled, lora_bt):
    v, d = main_weight.shape
    r = lora_a_scaled.shape[1]
    block = 1000
    grid = v // block
    return pl.pallas_call(
        _merge_body,
        grid=(grid,),
        in_specs=[
            pl.BlockSpec((block, d), lambda i: (i, 0)),
            pl.BlockSpec((block, r), lambda i: (i, 0)),
            pl.BlockSpec((r, d), lambda i: (0, 0)),
        ],
        out_specs=pl.BlockSpec((block, d), lambda i: (i, 0)),
        out_shape=jax.ShapeDtypeStruct((v, d), jnp.float32),
    )(main_weight, lora_a_scaled, lora_bt)


def _make_gather(nchunk, d):
    mesh = plsc.VectorSubcoreMesh(
        core_axis_name="c", subcore_axis_name="s", num_cores=_NC, num_subcores=_NS
    )

    @functools.partial(
        pl.kernel,
        out_type=jax.ShapeDtypeStruct((_NW, nchunk, _CHUNK, d), jnp.float32),
        mesh=mesh,
        scratch_types=[
            pltpu.VMEM((nchunk, _CHUNK), jnp.int32),
            pltpu.VMEM((_CHUNK, d), jnp.float32),
            pltpu.SemaphoreType.DMA,
        ],
    )
    def gather(table_hbm, idx_hbm, out_hbm, idx_v, rows_v, sem):
        wid = lax.axis_index("s") * _NC + lax.axis_index("c")
        pltpu.sync_copy(idx_hbm.at[wid], idx_v)

        def chunk(j, carry):
            pltpu.async_copy(table_hbm.at[idx_v.at[j]], rows_v, sem).wait()
            pltpu.sync_copy(rows_v, out_hbm.at[wid].at[j])
            return carry

        lax.fori_loop(0, nchunk, chunk, 0)

    return gather


def kernel(idx, main_weight, lora_A, lora_B):
    b, l = idx.shape
    v, d = main_weight.shape
    rank = lora_A.shape[1]
    alpha = 32.0
    scale = alpha / rank

    merged = _merged_table(main_weight, lora_A * scale, lora_B.T)

    n = b * l
    assert n % (_NW * _CHUNK) == 0
    nchunk = n // (_NW * _CHUNK)
    idx3 = idx.astype(jnp.int32).reshape(_NW, nchunk, _CHUNK)
    out = _make_gather(nchunk, d)(merged, idx3)
    return out.reshape(b, l, d)


# speedup vs baseline: 3.9505x; 3.9505x over previous
import functools

import jax
import jax.numpy as jnp
from jax import lax
from jax.experimental import pallas as pl
from jax.experimental.pallas import tpu as pltpu
from jax.experimental.pallas import tpu_sc as plsc

_NC = 2
_NS = 16
_NW = _NC * _NS
_CHUNK = 128


def _merge_body(scale, main_ref, a_ref, bt_ref, out_ref):
    out_ref[...] = main_ref[...] + scale * jnp.dot(
        a_ref[...], bt_ref[...], preferred_element_type=jnp.float32
    )


def _merged_table(main_weight, lora_a, lora_bt, scale):
    v, d = main_weight.shape
    r = lora_a.shape[1]
    block = 1000
    grid = v // block
    return pl.pallas_call(
        functools.partial(_merge_body, scale),
        grid=(grid,),
        in_specs=[
            pl.BlockSpec((block, d), lambda i: (i, 0)),
            pl.BlockSpec((block, r), lambda i: (i, 0)),
            pl.BlockSpec((r, d), lambda i: (0, 0)),
        ],
        out_specs=pl.BlockSpec((block, d), lambda i: (i, 0)),
        out_shape=jax.ShapeDtypeStruct((v, d), jnp.float32),
    )(main_weight, lora_a, lora_bt)


def _make_gather(nchunk, d):
    mesh = plsc.VectorSubcoreMesh(
        core_axis_name="c", subcore_axis_name="s", num_cores=_NC, num_subcores=_NS
    )

    @functools.partial(
        pl.kernel,
        out_type=jax.ShapeDtypeStruct((_NW, nchunk, _CHUNK, d), jnp.float32),
        mesh=mesh,
        scratch_types=[
            pltpu.VMEM((nchunk, _CHUNK), jnp.int32),
            pltpu.VMEM((_CHUNK, d), jnp.float32),
            pltpu.SemaphoreType.DMA,
        ],
    )
    def gather(table_hbm, idx_hbm, out_hbm, idx_v, rows_v, sem):
        wid = lax.axis_index("s") * _NC + lax.axis_index("c")
        pltpu.sync_copy(idx_hbm.at[wid], idx_v)

        def chunk(j, carry):
            pltpu.async_copy(table_hbm.at[idx_v.at[j]], rows_v, sem).wait()
            pltpu.sync_copy(rows_v, out_hbm.at[wid].at[j])
            return carry

        lax.fori_loop(0, nchunk, chunk, 0)

    return gather


def kernel(idx, main_weight, lora_A, lora_B):
    b, l = idx.shape
    v, d = main_weight.shape
    rank = lora_A.shape[1]
    alpha = 32.0
    scale = alpha / rank

    merged = _merged_table(main_weight, lora_A, lora_B.T, scale)

    n = b * l
    assert n % (_NW * _CHUNK) == 0
    nchunk = n // (_NW * _CHUNK)
    idx3 = idx.astype(jnp.int32).reshape(_NW, nchunk, _CHUNK)
    out = _make_gather(nchunk, d)(merged, idx3)
    return out.reshape(b, l, d)
